# Initial kernel scaffold; baseline (speedup 1.0000x reference)
import jax
import jax.numpy as jnp
from jax.experimental import pallas as pl


def kernel(words, pos, word_table, pos_table):
    raise NotImplementedError("write your pallas kernel here")


# speedup vs baseline: 2.0832x; 2.0832x over previous
import functools

import jax
import jax.numpy as jnp
from jax import lax
from jax.experimental import pallas as pl
from jax.experimental.pallas import tpu as pltpu
from jax.experimental.pallas import tpu_sc as plsc

NC = 2
NS = 16
NW = NC * NS

WDIM = 64
PDIM = 32

GATHER_W = 128
GATHERS_PER_CHUNK = 8
CHUNK = GATHER_W * GATHERS_PER_CHUNK


@functools.partial(jax.jit, static_argnums=(4,))
def _embed(words_2d, pos_2d, word_table, pos_table, n):
  per_w = n // NW
  iters = per_w // CHUNK
  rows_per_chunk = CHUNK // GATHER_W

  mesh = plsc.VectorSubcoreMesh(core_axis_name="c", subcore_axis_name="s")

  def body(words_hbm, pos_hbm, wtab_hbm, ptab_hbm, out_w_hbm, out_p_hbm,
           idx_w, idx_p, rows_w, rows_p, sem_w, sem_p):
    wid = lax.axis_index("s") * NC + lax.axis_index("c")
    base = wid * per_w

    def step(i, carry):
      off = pl.multiple_of(base + i * CHUNK, CHUNK)
      idx_row = pl.multiple_of(off // GATHER_W, rows_per_chunk)
      pltpu.sync_copy(words_hbm.at[pl.ds(idx_row, rows_per_chunk)], idx_w)
      pltpu.sync_copy(pos_hbm.at[pl.ds(idx_row, rows_per_chunk)], idx_p)
      copies = []
      for j in range(rows_per_chunk):
        copies.append(pltpu.async_copy(
            wtab_hbm.at[idx_w.at[j]],
            rows_w.at[pl.ds(j * GATHER_W, GATHER_W)], sem_w))
        copies.append(pltpu.async_copy(
            ptab_hbm.at[idx_p.at[j]],
            rows_p.at[pl.ds(j * GATHER_W, GATHER_W)], sem_p))
      for c in copies:
        c.wait()
      pltpu.sync_copy(rows_w, out_w_hbm.at[pl.ds(off, CHUNK)])
      pltpu.sync_copy(rows_p, out_p_hbm.at[pl.ds(off, CHUNK)])
      return carry

    lax.fori_loop(0, iters, step, 0)

  run = pl.kernel(
      body,
      out_type=(
          jax.ShapeDtypeStruct((n, WDIM), jnp.float32),
          jax.ShapeDtypeStruct((n, PDIM), jnp.float32),
      ),
      mesh=mesh,
      compiler_params=pltpu.CompilerParams(use_tc_tiling_on_sc=False),
      scratch_types=[
          pltpu.VMEM((CHUNK // GATHER_W, GATHER_W), jnp.int32),
          pltpu.VMEM((CHUNK // GATHER_W, GATHER_W), jnp.int32),
          pltpu.VMEM((CHUNK, WDIM), jnp.float32),
          pltpu.VMEM((CHUNK, PDIM), jnp.float32),
          pltpu.SemaphoreType.DMA,
          pltpu.SemaphoreType.DMA,
      ],
  )
  return run(words_2d, pos_2d, word_table, pos_table)


def kernel(words, pos, word_table, pos_table):
  B, L = words.shape
  n = B * L
  words_2d = words.reshape(n // GATHER_W, GATHER_W).astype(jnp.int32)
  pos_2d = pos.reshape(n // GATHER_W, GATHER_W).astype(jnp.int32)
  out_w, out_p = _embed(words_2d, pos_2d, word_table, pos_table, n)
  return (out_w.reshape(B, L, WDIM), out_p.reshape(B, L, PDIM))


# speedup vs baseline: 2.0864x; 1.0016x over previous
import functools

import jax
import jax.numpy as jnp
from jax import lax
from jax.experimental import pallas as pl
from jax.experimental.pallas import tpu as pltpu
from jax.experimental.pallas import tpu_sc as plsc

NC = 2
NS = 16
NW = NC * NS

WDIM = 64
PDIM = 32

GATHER_W = 128
ROWS_PER_CHUNK = 4
CHUNK = GATHER_W * ROWS_PER_CHUNK


@functools.partial(jax.jit, static_argnums=(4,))
def _embed(words_3d, pos_3d, word_table, pos_table, n):
  per_w = n // NW
  iters = per_w // CHUNK
  assert iters % 2 == 0

  mesh = plsc.VectorSubcoreMesh(core_axis_name="c", subcore_axis_name="s")

  def body(words_hbm, pos_hbm, wtab_hbm, ptab_hbm, out_w_hbm, out_p_hbm,
           idx_w, idx_p, rows_w, rows_p,
           sem_iw0, sem_iw1, sem_ip0, sem_ip1,
           sem_g, sem_ww0, sem_ww1, sem_wp0, sem_wp1):
    wid = lax.axis_index("s") * NC + lax.axis_index("c")
    base_chunk = wid * iters
    sem_iw = (sem_iw0, sem_iw1)
    sem_ip = (sem_ip0, sem_ip1)
    sem_ww = (sem_ww0, sem_ww1)
    sem_wp = (sem_wp0, sem_wp1)

    def idx_load(c, p):
      pltpu.async_copy(words_hbm.at[c], idx_w.at[p], sem_iw[p])
      pltpu.async_copy(pos_hbm.at[c], idx_p.at[p], sem_ip[p])

    def idx_wait(p):
      pltpu.make_async_copy(words_hbm.at[0], idx_w.at[p], sem_iw[p]).wait()
      pltpu.make_async_copy(pos_hbm.at[0], idx_p.at[p], sem_ip[p]).wait()

    def wb_wait(p):
      pltpu.make_async_copy(
          rows_w.at[p], out_w_hbm.at[pl.ds(0, CHUNK)], sem_ww[p]).wait()
      pltpu.make_async_copy(
          rows_p.at[p], out_p_hbm.at[pl.ds(0, CHUNK)], sem_wp[p]).wait()

    idx_load(base_chunk, 0)
    idx_load(base_chunk + 1, 1)

    def step(t, carry):
      for p in (0, 1):
        c = base_chunk + 2 * t + p
        idx_wait(p)
        @pl.when(t > 0)
        def _():
          wb_wait(p)
        copies = []
        for j in range(ROWS_PER_CHUNK):
          copies.append(pltpu.async_copy(
              wtab_hbm.at[idx_w.at[p].at[j]],
              rows_w.at[p].at[pl.ds(j * GATHER_W, GATHER_W)], sem_g))
          copies.append(pltpu.async_copy(
              ptab_hbm.at[idx_p.at[p].at[j]],
              rows_p.at[p].at[pl.ds(j * GATHER_W, GATHER_W)], sem_g))
        for cp in copies:
          cp.wait()
        @pl.when(2 * t + p + 2 < iters)
        def _():
          idx_load(c + 2, p)
        off = pl.multiple_of(c * CHUNK, CHUNK)
        pltpu.async_copy(rows_w.at[p], out_w_hbm.at[pl.ds(off, CHUNK)],
                         sem_ww[p])
        pltpu.async_copy(rows_p.at[p], out_p_hbm.at[pl.ds(off, CHUNK)],
                         sem_wp[p])
      return carry

    lax.fori_loop(0, iters // 2, step, 0)
    wb_wait(0)
    wb_wait(1)

  run = pl.kernel(
      body,
      out_type=(
          jax.ShapeDtypeStruct((n, WDIM), jnp.float32),
          jax.ShapeDtypeStruct((n, PDIM), jnp.float32),
      ),
      mesh=mesh,
      compiler_params=pltpu.CompilerParams(use_tc_tiling_on_sc=False),
      scratch_types=[
          pltpu.VMEM((2, ROWS_PER_CHUNK, GATHER_W), jnp.int32),
          pltpu.VMEM((2, ROWS_PER_CHUNK, GATHER_W), jnp.int32),
          pltpu.VMEM((2, CHUNK, WDIM), jnp.float32),
          pltpu.VMEM((2, CHUNK, PDIM), jnp.float32),
          pltpu.SemaphoreType.DMA,
          pltpu.SemaphoreType.DMA,
          pltpu.SemaphoreType.DMA,
          pltpu.SemaphoreType.DMA,
          pltpu.SemaphoreType.DMA,
          pltpu.SemaphoreType.DMA,
          pltpu.SemaphoreType.DMA,
          pltpu.SemaphoreType.DMA,
          pltpu.SemaphoreType.DMA,
      ],
  )
  return run(words_3d, pos_3d, word_table, pos_table)


def kernel(words, pos, word_table, pos_table):
  B, L = words.shape
  n = B * L
  words_3d = words.reshape(n // CHUNK, ROWS_PER_CHUNK, GATHER_W).astype(
      jnp.int32)
  pos_3d = pos.reshape(n // CHUNK, ROWS_PER_CHUNK, GATHER_W).astype(jnp.int32)
  out_w, out_p = _embed(words_3d, pos_3d, word_table, pos_table, n)
  return (out_w.reshape(B, L, WDIM), out_p.reshape(B, L, PDIM))
